# baseline (speedup 1.0000x reference)
import functools

import jax
import jax.numpy as jnp
from jax import lax
from jax.experimental import pallas as pl
from jax.experimental.pallas import tpu as pltpu
from jax.experimental.pallas import tpu_sc as plsc

LATENT_DIM = 64
LANES = 16
CHUNK = 128


@jax.jit
def _mf_forward(u_hi, i_hi, u_par, i_par, users_packed, items_packed):
    batch = u_par.shape[0]
    info = plsc.get_sparse_core_info()
    nw = info.num_cores * info.num_subcores
    bpw = batch // nw
    n_chunks = bpw // CHUNK
    groups_per_chunk = CHUNK // LANES
    mesh = plsc.VectorSubcoreMesh(core_axis_name="c", subcore_axis_name="s")

    @functools.partial(
        pl.kernel,
        out_type=jax.ShapeDtypeStruct((batch,), jnp.float32),
        mesh=mesh,
        compiler_params=pltpu.CompilerParams(needs_layout_passes=False),
        scratch_types=[
            pltpu.VMEM((n_chunks, CHUNK), jnp.int32),
            pltpu.VMEM((n_chunks, CHUNK), jnp.int32),
            pltpu.VMEM((bpw,), jnp.int32),
            pltpu.VMEM((bpw,), jnp.int32),
            pltpu.VMEM((CHUNK, 2 * LATENT_DIM), jnp.float32),
            pltpu.VMEM((CHUNK, 2 * LATENT_DIM), jnp.float32),
            pltpu.VMEM((CHUNK, 2 * LATENT_DIM), jnp.float32),
            pltpu.VMEM((CHUNK, 2 * LATENT_DIM), jnp.float32),
            pltpu.VMEM((bpw,), jnp.float32),
            pltpu.SemaphoreType.DMA,
            pltpu.SemaphoreType.DMA,
        ],
    )
    def kern(uh_hbm, ih_hbm, up_hbm, ip_hbm, users_hbm, items_hbm, out_hbm,
             uh_v, ih_v, up_v, ip_v, ub0, ub1, ib0, ib1, out_v, sem0, sem1):
        wid = lax.axis_index("s") * info.num_cores + lax.axis_index("c")
        base = wid * bpw

        pltpu.sync_copy(uh_hbm.at[pl.ds(wid * n_chunks, n_chunks)], uh_v)
        pltpu.sync_copy(ih_hbm.at[pl.ds(wid * n_chunks, n_chunks)], ih_v)
        pltpu.sync_copy(up_hbm.at[pl.ds(base, bpw)], up_v)
        pltpu.sync_copy(ip_hbm.at[pl.ds(base, bpw)], ip_v)

        ubufs = [ub0, ub1]
        ibufs = [ib0, ib1]
        sems = [sem0, sem1]

        def fire(j):
            s = sems[j % 2]
            return (
                pltpu.async_copy(users_hbm.at[uh_v.at[j]], ubufs[j % 2], s),
                pltpu.async_copy(items_hbm.at[ih_v.at[j]], ibufs[j % 2], s),
            )

        lanes_iota = lax.iota(jnp.int32, LANES)
        pending = fire(0)

        for j in range(n_chunks):
            for c in pending:
                c.wait()
            if j + 1 < n_chunks:
                nxt = fire(j + 1)
            ub, ib = ubufs[j % 2], ibufs[j % 2]

            def group_body(gg, _):
                row0 = j * CHUNK + gg * LANES
                rows_local = gg * LANES + lanes_iota
                ucols = up_v[pl.ds(row0, LANES)] * LATENT_DIM
                icols = ip_v[pl.ds(row0, LANES)] * LATENT_DIM
                acc = (plsc.load_gather(ub, [rows_local, ucols])
                       * plsc.load_gather(ib, [rows_local, icols]))
                for d in range(1, LATENT_DIM):
                    acc = acc + (plsc.load_gather(ub, [rows_local, ucols + d])
                                 * plsc.load_gather(ib, [rows_local, icols + d]))
                out_v[pl.ds(row0, LANES)] = acc
                return 0

            lax.fori_loop(0, groups_per_chunk, group_body, 0)
            if j + 1 < n_chunks:
                pending = nxt

        pltpu.sync_copy(out_v, out_hbm.at[pl.ds(base, bpw)])

    return kern(u_hi, i_hi, u_par, i_par, users_packed, items_packed)


def kernel(x, users_weight, items_weight):
    x32 = x.astype(jnp.int32)
    u = x32[:, 0]
    it = x32[:, 1]
    u_hi = (u >> 1).reshape(-1, CHUNK)
    i_hi = (it >> 1).reshape(-1, CHUNK)
    u_par = u & 1
    i_par = it & 1
    users_packed = users_weight.reshape(-1, 2 * LATENT_DIM)
    items_packed = items_weight.reshape(-1, 2 * LATENT_DIM)
    return _mf_forward(u_hi, i_hi, u_par, i_par, users_packed, items_packed)
